# baseline (speedup 1.0000x reference)
import jax
import jax.numpy as jnp
from jax.experimental import pallas as pl
from jax.experimental.pallas import tpu as pltpu

_B, _C, _H, _W = 16, 256, 32, 32


def _body(row_ref, col_ref, out_ref):
    hw = _H * _W
    iota_r = jax.lax.broadcasted_iota(jnp.int32, (_W, hw), 0)
    iota_j = jax.lax.broadcasted_iota(jnp.int32, (_W, hw), 1)
    sel_x = ((iota_j & (_W - 1)) == iota_r).astype(jnp.float32)
    sel_y = ((iota_j >> 5) == iota_r).astype(jnp.float32)
    dn = (((0,), (0,)), ((), ()))
    out_ref[0, :_C] = jax.lax.dot_general(
        col_ref[...], sel_x, dn, preferred_element_type=jnp.float32)
    out_ref[0, _C:] = jax.lax.dot_general(
        row_ref[...], sel_y, dn, preferred_element_type=jnp.float32)


def kernel(mask, row_embed, col_embed):
    b = mask.shape[0]
    h, w = mask.shape[-2], mask.shape[-1]
    c = row_embed.shape[-1]
    out = pl.pallas_call(
        _body,
        grid=(b,),
        in_specs=[
            pl.BlockSpec((h, c), lambda i: (0, 0)),
            pl.BlockSpec((w, c), lambda i: (0, 0)),
        ],
        out_specs=pl.BlockSpec((1, 2 * c, h * w), lambda i: (i, 0, 0)),
        out_shape=jax.ShapeDtypeStruct((b, 2 * c, h * w), jnp.float32),
    )(row_embed[:h], col_embed[:w])
    return out.reshape(b, 2 * c, h, w)
